# baseline (speedup 1.0000x reference)
import functools

import jax
import jax.numpy as jnp
import numpy as np
from jax import lax
from jax.experimental import pallas as pl
from jax.experimental.pallas import tpu as pltpu
from jax.experimental.pallas import tpu_sc as plsc

N = 100000
D = 256
G = 1024
NW = 32
ROWS_PER_W = N // NW
HB = 128
NHB = 26
WLEN = NHB * HB
NPH = 2
NHBP = NHB // NPH
SB = 128
NSB = NHBP * 2
SROWS_W = NSB * SB
G2 = 2 * G
GPAD = G2 + 8
CL = 16
ZSTRIPE = G2 // 16

_STARTS = [w * ROWS_PER_W // HB * HB for w in range(NW)]
assert _STARTS[-1] + WLEN == 100096 and (100096 - N) < HB


def _retile_body(h_ref, out_ref):
    x = h_ref[...]
    a = x[:, :128].reshape(16, 8, 128)
    b = x[:, 128:].reshape(16, 8, 128)
    out_ref[...] = jnp.concatenate([a, b], axis=1).reshape(2 * HB, 128)


def _retile(h, phase):
    return pl.pallas_call(
        _retile_body,
        grid=(NW, NHBP),
        in_specs=[pl.BlockSpec(
            (HB, D),
            lambda w, j: (w * ROWS_PER_W // HB + phase * NHBP + j, 0))],
        out_specs=pl.BlockSpec((2 * HB, 128), lambda w, j: (w * NHBP + j, 0)),
        out_shape=jax.ShapeDtypeStruct((NW * SROWS_W, 128), jnp.float32),
    )(h)


_SC_MESH = plsc.VectorSubcoreMesh(core_axis_name="c", subcore_axis_name="s")


@functools.partial(
    pl.kernel,
    mesh=_SC_MESH,
    out_type=[
        jax.ShapeDtypeStruct((2 * GPAD, 128), jnp.float32),
        jax.ShapeDtypeStruct((2 * GPAD, CL), jnp.float32),
    ],
    scratch_types=[
        pltpu.VMEM((NSB, SB), jnp.int32),
        pltpu.VMEM((SB, 128), jnp.float32),
        pltpu.VMEM((SB, 128), jnp.float32),
        pltpu.VMEM((SB, 128), jnp.float32),
        pltpu.VMEM((SB, CL), jnp.float32),
        pltpu.VMEM_SHARED((GPAD, 128), jnp.float32),
        pltpu.VMEM_SHARED((GPAD, CL), jnp.float32),
        pltpu.SemaphoreType.DMA,
        pltpu.SemaphoreType.DMA,
        pltpu.SemaphoreType.DMA,
        pltpu.SemaphoreType.DMA,
        pltpu.SemaphoreType.DMA,
        pltpu.SemaphoreType.DMA,
    ],
    compiler_params=pltpu.CompilerParams(use_tc_tiling_on_sc=False),
)
def _seg_sum_sc(ids_hbm, h4_hbm, zsum_hbm, zcnt_hbm, ones_hbm,
                sums_hbm, cnts_hbm,
                ids_v, buf0, buf1, buf2, ones_v, acc_s, cnt_s,
                ld0, ld1, ld2, st0, st1, st2):
    cid = lax.axis_index("c")
    sid = lax.axis_index("s")
    wid = sid * 2 + cid
    bufs = (buf0, buf1, buf2)
    lds = (ld0, ld1, ld2)
    sts = (st0, st1, st2)

    pltpu.sync_copy(ids_hbm.at[wid], ids_v)
    pltpu.sync_copy(ones_hbm, ones_v)
    pltpu.sync_copy(zsum_hbm.at[pl.ds(sid * ZSTRIPE, ZSTRIPE)],
                    acc_s.at[pl.ds(sid * ZSTRIPE, ZSTRIPE)])
    pltpu.sync_copy(zcnt_hbm.at[pl.ds(sid * ZSTRIPE, ZSTRIPE)],
                    cnt_s.at[pl.ds(sid * ZSTRIPE, ZSTRIPE)])
    plsc.subcore_barrier()

    base = wid * SROWS_W

    def h_src(b):
        return h4_hbm.at[pl.ds(base + b * SB, SB)]

    def start_scat(b, k):
        pltpu.async_copy(bufs[k], acc_s.at[ids_v.at[b]], sts[k], add=True)
        pltpu.async_copy(ones_v, cnt_s.at[ids_v.at[b]], sts[k], add=True)

    def wait_scat(b, k):
        pltpu.make_async_copy(bufs[k], acc_s.at[ids_v.at[b]], sts[k]).wait()
        pltpu.make_async_copy(ones_v, cnt_s.at[ids_v.at[b]], sts[k]).wait()

    pltpu.async_copy(h_src(0), bufs[0], lds[0])

    def group(g, carry):
        for k in range(3):
            b = g * 3 + k
            kn = (k + 1) % 3
            @pl.when(b >= 2)
            def _():
                wait_scat(b - 2, kn)
            pltpu.async_copy(h_src(b + 1), bufs[kn], lds[kn])
            pltpu.make_async_copy(h_src(b), bufs[k], lds[k]).wait()
            start_scat(b, k)
        return carry

    ngrp = (NSB - 2) // 3
    lax.fori_loop(0, ngrp, group, 0)

    for b in range(3 * ngrp, NSB):
        k = b % 3
        if b + 1 < NSB:
            kn = (k + 1) % 3
            wait_scat(b - 2, kn)
            pltpu.async_copy(h_src(b + 1), bufs[kn], lds[kn])
        pltpu.make_async_copy(h_src(b), bufs[k], lds[k]).wait()
        start_scat(b, k)
    wait_scat(NSB - 3, (NSB - 3) % 3)
    wait_scat(NSB - 2, (NSB - 2) % 3)
    wait_scat(NSB - 1, (NSB - 1) % 3)
    plsc.subcore_barrier()

    pltpu.sync_copy(acc_s.at[pl.ds(sid * ZSTRIPE, ZSTRIPE)],
                    sums_hbm.at[pl.ds(cid * GPAD + sid * ZSTRIPE, ZSTRIPE)])
    pltpu.sync_copy(cnt_s.at[pl.ds(sid * ZSTRIPE, ZSTRIPE)],
                    cnts_hbm.at[pl.ds(cid * GPAD + sid * ZSTRIPE, ZSTRIPE)])


def _mlp_body(s0_ref, s1_ref, c0_ref, c1_ref, fcw_ref, fcb_ref,
              clsw_ref, clsb_ref, out_ref):
    sums = (s0_ref[0] + s0_ref[1] + s1_ref[0] + s1_ref[1])
    cnt = (c0_ref[0] + c0_ref[1] + c1_ref[0] + c1_ref[1])
    cnt0 = jnp.maximum(cnt[:, 0:1], 1.0)
    gf = sums / cnt0
    hidden = jnp.maximum(jnp.dot(gf, fcw_ref[...]) + fcb_ref[...], 0.0)
    out_ref[...] = jnp.dot(hidden, clsw_ref[...]) + clsb_ref[...]


_OWNED = np.stack([
    (np.arange(s, s + WLEN) >= w * ROWS_PER_W)
    & (np.arange(s, s + WLEN) < (w + 1) * ROWS_PER_W)
    for w, s in enumerate(_STARTS)
])


def _build_ids(graph_ids, phase):
    gid = graph_ids.astype(jnp.int32)
    gidp = jnp.pad(gid, (0, _STARTS[-1] + WLEN - N), constant_values=G)
    wins = jnp.stack([lax.slice(gidp, (s,), (s + WLEN,))
                      for s in _STARTS])
    base = jnp.where(_OWNED, wins * 2, G2)
    lo = phase * NHBP * HB
    base = base[:, lo:lo + NHBP * HB]
    base = base.reshape(NW, NHBP, 16, 1, 8)
    idx = base + jnp.arange(2, dtype=jnp.int32).reshape(1, 1, 1, 2, 1)
    return idx.reshape(NW, NSB, SB)


def kernel(h, graph_ids, fc_w, fc_b, cls_w, cls_b):
    zsum = jnp.zeros((G2, 128), jnp.float32)
    zcnt = jnp.zeros((G2, CL), jnp.float32)
    ones = jnp.ones((SB, CL), jnp.float32)

    parts = []
    for p in range(NPH):
        h4 = _retile(h, p)
        ids = _build_ids(graph_ids, p)
        parts.append(_seg_sum_sc(ids, h4, zsum, zcnt, ones))

    (s0, c0), (s1, c1) = parts
    s0 = s0.reshape(2, GPAD // 2, D)[:, :G, :]
    s1 = s1.reshape(2, GPAD // 2, D)[:, :G, :]
    c0 = c0.reshape(2, GPAD // 2, 2, CL)[:, :G, 0, :]
    c1 = c1.reshape(2, GPAD // 2, 2, CL)[:, :G, 0, :]
    out = pl.pallas_call(
        _mlp_body,
        out_shape=jax.ShapeDtypeStruct((G, 16), jnp.float32),
    )(s0, s1, c0, c1,
      fc_w, fc_b.reshape(1, 512), cls_w, cls_b.reshape(1, 16))
    return out


# speedup vs baseline: 2.8979x; 2.8979x over previous
import functools

import jax
import jax.numpy as jnp
from jax import lax
from jax.experimental import pallas as pl
from jax.experimental.pallas import tpu as pltpu
from jax.experimental.pallas import tpu_sc as plsc

N = 100000
D = 256
G = 1024
NW = 32
NPH = 2
SLAB = 2560
NSLAB = 20
PH_ROWS = SLAB * NSLAB
NPAD = NPH * PH_ROWS
ROWS_PW = PH_ROWS // NW
SB = 128
NSB = ROWS_PW * 2 // SB
SROWS_W = NSB * SB
G2 = 2 * G
GPAD = G2 + 8
CL = 16
ZSTRIPE = G2 // 16


def _retile_body(h_ref, out_ref):
    x = h_ref[...]
    a = x[:, :128].reshape(SLAB // 8, 8, 128)
    b = x[:, 128:].reshape(SLAB // 8, 8, 128)
    out_ref[...] = jnp.concatenate([a, b], axis=1).reshape(2 * SLAB, 128)


def _retile(h, phase):
    return pl.pallas_call(
        _retile_body,
        grid=(NSLAB,),
        in_specs=[pl.BlockSpec((SLAB, D), lambda i: (phase * NSLAB + i, 0))],
        out_specs=pl.BlockSpec((2 * SLAB, 128), lambda i: (i, 0)),
        out_shape=jax.ShapeDtypeStruct((2 * PH_ROWS, 128), jnp.float32),
    )(h)


_SC_MESH = plsc.VectorSubcoreMesh(core_axis_name="c", subcore_axis_name="s")


@functools.partial(
    pl.kernel,
    mesh=_SC_MESH,
    out_type=[
        jax.ShapeDtypeStruct((2 * GPAD, 128), jnp.float32),
        jax.ShapeDtypeStruct((2 * GPAD, CL), jnp.float32),
    ],
    scratch_types=[
        pltpu.VMEM((NSB, SB), jnp.int32),
        pltpu.VMEM((SB, 128), jnp.float32),
        pltpu.VMEM((SB, 128), jnp.float32),
        pltpu.VMEM((SB, 128), jnp.float32),
        pltpu.VMEM((SB, CL), jnp.float32),
        pltpu.VMEM_SHARED((GPAD, 128), jnp.float32),
        pltpu.VMEM_SHARED((GPAD, CL), jnp.float32),
        pltpu.SemaphoreType.DMA,
        pltpu.SemaphoreType.DMA,
        pltpu.SemaphoreType.DMA,
        pltpu.SemaphoreType.DMA,
        pltpu.SemaphoreType.DMA,
        pltpu.SemaphoreType.DMA,
    ],
    compiler_params=pltpu.CompilerParams(use_tc_tiling_on_sc=False),
)
def _seg_sum_sc(ids_hbm, h4_hbm, zsum_hbm, zcnt_hbm, ones_hbm,
                sums_hbm, cnts_hbm,
                ids_v, buf0, buf1, buf2, ones_v, acc_s, cnt_s,
                ld0, ld1, ld2, st0, st1, st2):
    cid = lax.axis_index("c")
    sid = lax.axis_index("s")
    wid = sid * 2 + cid
    bufs = (buf0, buf1, buf2)
    lds = (ld0, ld1, ld2)
    sts = (st0, st1, st2)

    pltpu.sync_copy(ids_hbm.at[wid], ids_v)
    pltpu.sync_copy(ones_hbm, ones_v)
    pltpu.sync_copy(zsum_hbm.at[pl.ds(sid * ZSTRIPE, ZSTRIPE)],
                    acc_s.at[pl.ds(sid * ZSTRIPE, ZSTRIPE)])
    pltpu.sync_copy(zcnt_hbm.at[pl.ds(sid * ZSTRIPE, ZSTRIPE)],
                    cnt_s.at[pl.ds(sid * ZSTRIPE, ZSTRIPE)])
    plsc.subcore_barrier()

    base = wid * SROWS_W

    def h_src(b):
        return h4_hbm.at[pl.ds(base + b * SB, SB)]

    def start_scat(b, k):
        pltpu.async_copy(bufs[k], acc_s.at[ids_v.at[b]], sts[k], add=True)
        pltpu.async_copy(ones_v, cnt_s.at[ids_v.at[b]], sts[k], add=True)

    def wait_scat(b, k):
        pltpu.make_async_copy(bufs[k], acc_s.at[ids_v.at[b]], sts[k]).wait()
        pltpu.make_async_copy(ones_v, cnt_s.at[ids_v.at[b]], sts[k]).wait()

    pltpu.async_copy(h_src(0), bufs[0], lds[0])

    def group(g, carry):
        for k in range(3):
            b = g * 3 + k
            kn = (k + 1) % 3
            @pl.when(b >= 2)
            def _():
                wait_scat(b - 2, kn)
            pltpu.async_copy(h_src(b + 1), bufs[kn], lds[kn])
            pltpu.make_async_copy(h_src(b), bufs[k], lds[k]).wait()
            start_scat(b, k)
        return carry

    ngrp = (NSB - 2) // 3
    lax.fori_loop(0, ngrp, group, 0)

    for b in range(3 * ngrp, NSB):
        k = b % 3
        if b + 1 < NSB:
            kn = (k + 1) % 3
            wait_scat(b - 2, kn)
            pltpu.async_copy(h_src(b + 1), bufs[kn], lds[kn])
        pltpu.make_async_copy(h_src(b), bufs[k], lds[k]).wait()
        start_scat(b, k)
    wait_scat(NSB - 3, (NSB - 3) % 3)
    wait_scat(NSB - 2, (NSB - 2) % 3)
    wait_scat(NSB - 1, (NSB - 1) % 3)
    plsc.subcore_barrier()

    pltpu.sync_copy(acc_s.at[pl.ds(sid * ZSTRIPE, ZSTRIPE)],
                    sums_hbm.at[pl.ds(cid * GPAD + sid * ZSTRIPE, ZSTRIPE)])
    pltpu.sync_copy(cnt_s.at[pl.ds(sid * ZSTRIPE, ZSTRIPE)],
                    cnts_hbm.at[pl.ds(cid * GPAD + sid * ZSTRIPE, ZSTRIPE)])


def _mlp_body(s0_ref, s1_ref, c0_ref, c1_ref, fcw_ref, fcb_ref,
              clsw_ref, clsb_ref, out_ref):
    sums = (s0_ref[0] + s0_ref[1] + s1_ref[0] + s1_ref[1])
    cnt = (c0_ref[0] + c0_ref[1] + c1_ref[0] + c1_ref[1])
    cnt0 = jnp.maximum(cnt[:, 0:1], 1.0)
    gf = sums / cnt0
    hidden = jnp.maximum(jnp.dot(gf, fcw_ref[...]) + fcb_ref[...], 0.0)
    out_ref[...] = jnp.dot(hidden, clsw_ref[...]) + clsb_ref[...]


def _build_ids(graph_ids):
    gid = graph_ids.astype(jnp.int32)
    gidp = jnp.pad(gid, (0, NPAD - N), constant_values=G)
    base = (gidp * 2).reshape(NPH, NW, NSB, 8, 1, 8)
    idx = base + jnp.arange(2, dtype=jnp.int32).reshape(1, 1, 1, 1, 2, 1)
    return idx.reshape(NPH, NW, NSB, SB)


def kernel(h, graph_ids, fc_w, fc_b, cls_w, cls_b):
    zsum = jnp.zeros((G2, 128), jnp.float32)
    zcnt = jnp.zeros((G2, CL), jnp.float32)
    ones = jnp.ones((SB, CL), jnp.float32)
    ids = _build_ids(graph_ids)

    parts = []
    for p in range(NPH):
        h4 = _retile(h, p)
        parts.append(_seg_sum_sc(ids[p], h4, zsum, zcnt, ones))

    (s0, c0), (s1, c1) = parts
    s0 = s0.reshape(2, GPAD // 2, D)[:, :G, :]
    s1 = s1.reshape(2, GPAD // 2, D)[:, :G, :]
    c0 = c0.reshape(2, GPAD // 2, 2, CL)[:, :G, 0, :]
    c1 = c1.reshape(2, GPAD // 2, 2, CL)[:, :G, 0, :]
    out = pl.pallas_call(
        _mlp_body,
        out_shape=jax.ShapeDtypeStruct((G, 16), jnp.float32),
    )(s0, s1, c0, c1,
      fc_w, fc_b.reshape(1, 512), cls_w, cls_b.reshape(1, 16))
    return out


# speedup vs baseline: 2.9718x; 1.0255x over previous
import functools

import jax
import jax.numpy as jnp
from jax import lax
from jax.experimental import pallas as pl
from jax.experimental.pallas import tpu as pltpu
from jax.experimental.pallas import tpu_sc as plsc

N = 100000
D = 256
G = 1024
NW = 32
NPH = 2
SLAB = 2560
NSLAB = 20
PH_ROWS = SLAB * NSLAB
NPAD = NPH * PH_ROWS
ROWS_PW = PH_ROWS // NW
SB = 128
NSB = ROWS_PW * 2 // SB
SROWS_W = NSB * SB
G2 = 2 * G
GPAD = G2 + 8
CL = 16
ZSTRIPE = G2 // 16


def _retile_body(h_ref, out_ref):
    x = h_ref[...]
    a = x[:, :128].reshape(SLAB // 8, 8, 128)
    b = x[:, 128:].reshape(SLAB // 8, 8, 128)
    out_ref[...] = jnp.concatenate([a, b], axis=1).reshape(2 * SLAB, 128)


def _retile(h, phase):
    return pl.pallas_call(
        _retile_body,
        grid=(NSLAB,),
        in_specs=[pl.BlockSpec((SLAB, D), lambda i: (phase * NSLAB + i, 0))],
        out_specs=pl.BlockSpec((2 * SLAB, 128), lambda i: (i, 0)),
        out_shape=jax.ShapeDtypeStruct((2 * PH_ROWS, 128), jnp.float32),
    )(h)


_SC_MESH = plsc.VectorSubcoreMesh(core_axis_name="c", subcore_axis_name="s")


@functools.partial(
    pl.kernel,
    mesh=_SC_MESH,
    out_type=[
        jax.ShapeDtypeStruct((2 * GPAD, 128), jnp.float32),
        jax.ShapeDtypeStruct((2 * GPAD, CL), jnp.float32),
    ],
    scratch_types=[
        pltpu.VMEM((NSB, SB), jnp.int32),
        pltpu.VMEM((SB, 128), jnp.float32),
        pltpu.VMEM((SB, 128), jnp.float32),
        pltpu.VMEM((SB, 128), jnp.float32),
        pltpu.VMEM((SB, 128), jnp.float32),
        pltpu.VMEM((SB, 128), jnp.float32),
        pltpu.VMEM((SB, CL), jnp.float32),
        pltpu.VMEM_SHARED((GPAD, 128), jnp.float32),
        pltpu.VMEM_SHARED((GPAD, CL), jnp.float32),
        pltpu.SemaphoreType.DMA,
        pltpu.SemaphoreType.DMA,
        pltpu.SemaphoreType.DMA,
        pltpu.SemaphoreType.DMA,
        pltpu.SemaphoreType.DMA,
        pltpu.SemaphoreType.DMA,
        pltpu.SemaphoreType.DMA,
        pltpu.SemaphoreType.DMA,
        pltpu.SemaphoreType.DMA,
        pltpu.SemaphoreType.DMA,
        pltpu.SemaphoreType.DMA,
    ],
    compiler_params=pltpu.CompilerParams(use_tc_tiling_on_sc=False),
)
def _seg_sum_sc(ids_hbm, h4_hbm, zsum_hbm, zcnt_hbm, ones_hbm,
                sums_hbm, cnts_hbm,
                ids_v, buf0, buf1, buf2, buf3, buf4, ones_v, acc_s, cnt_s,
                ld0, ld1, ld2, ld3, ld4, st0, st1, st2, st3, st4, ctsem):
    cid = lax.axis_index("c")
    sid = lax.axis_index("s")
    wid = sid * 2 + cid
    bufs = (buf0, buf1, buf2, buf3, buf4)
    lds = (ld0, ld1, ld2, ld3, ld4)
    sts = (st0, st1, st2, st3, st4)

    pltpu.sync_copy(ids_hbm.at[wid], ids_v)
    pltpu.sync_copy(ones_hbm, ones_v)
    pltpu.sync_copy(zsum_hbm.at[pl.ds(sid * ZSTRIPE, ZSTRIPE)],
                    acc_s.at[pl.ds(sid * ZSTRIPE, ZSTRIPE)])
    pltpu.sync_copy(zcnt_hbm.at[pl.ds(sid * ZSTRIPE, ZSTRIPE)],
                    cnt_s.at[pl.ds(sid * ZSTRIPE, ZSTRIPE)])
    plsc.subcore_barrier()

    base = wid * SROWS_W

    def h_src(b):
        return h4_hbm.at[pl.ds(base + b * SB, SB)]

    def start_scat(b, k):
        pltpu.async_copy(bufs[k], acc_s.at[ids_v.at[b]], sts[k], add=True)

    def wait_scat(b, k):
        pltpu.make_async_copy(bufs[k], acc_s.at[ids_v.at[b]], sts[k]).wait()

    def cnt_fire(b, carry):
        pltpu.async_copy(ones_v, cnt_s.at[ids_v.at[b]], ctsem, add=True)
        return carry

    lax.fori_loop(0, NSB, cnt_fire, 0)

    pltpu.async_copy(h_src(0), bufs[0], lds[0])
    pltpu.async_copy(h_src(1), bufs[1], lds[1])

    def group(g, carry):
        for k in range(5):
            b = g * 5 + k
            kn = (k + 2) % 5
            @pl.when(b >= 3)
            def _():
                wait_scat(b - 3, kn)
            @pl.when(b + 2 < NSB)
            def _():
                pltpu.async_copy(h_src(b + 2), bufs[kn], lds[kn])
            pltpu.make_async_copy(h_src(b), bufs[k], lds[k]).wait()
            start_scat(b, k)
        return carry

    assert NSB % 5 == 0
    lax.fori_loop(0, NSB // 5, group, 0)

    wait_scat(NSB - 3, (NSB - 3) % 5)
    wait_scat(NSB - 2, (NSB - 2) % 5)
    wait_scat(NSB - 1, (NSB - 1) % 5)

    def cnt_drain(b, carry):
        pltpu.make_async_copy(ones_v, cnt_s.at[ids_v.at[b]], ctsem).wait()
        return carry

    lax.fori_loop(0, NSB, cnt_drain, 0)
    plsc.subcore_barrier()

    pltpu.sync_copy(acc_s.at[pl.ds(sid * ZSTRIPE, ZSTRIPE)],
                    sums_hbm.at[pl.ds(cid * GPAD + sid * ZSTRIPE, ZSTRIPE)])
    pltpu.sync_copy(cnt_s.at[pl.ds(sid * ZSTRIPE, ZSTRIPE)],
                    cnts_hbm.at[pl.ds(cid * GPAD + sid * ZSTRIPE, ZSTRIPE)])


def _mlp_body(s0_ref, s1_ref, c0_ref, c1_ref, fcw_ref, fcb_ref,
              clsw_ref, clsb_ref, out_ref):
    sums = (s0_ref[0] + s0_ref[1] + s1_ref[0] + s1_ref[1])
    cnt = (c0_ref[0] + c0_ref[1] + c1_ref[0] + c1_ref[1])
    cnt0 = jnp.maximum(cnt[:, 0:1], 1.0)
    gf = sums / cnt0
    hidden = jnp.maximum(jnp.dot(gf, fcw_ref[...]) + fcb_ref[...], 0.0)
    out_ref[...] = jnp.dot(hidden, clsw_ref[...]) + clsb_ref[...]


def _build_ids(graph_ids):
    gid = graph_ids.astype(jnp.int32)
    gidp = jnp.pad(gid, (0, NPAD - N), constant_values=G)
    base = (gidp * 2).reshape(NPH, NW, NSB, 8, 1, 8)
    idx = base + jnp.arange(2, dtype=jnp.int32).reshape(1, 1, 1, 1, 2, 1)
    return idx.reshape(NPH, NW, NSB, SB)


def kernel(h, graph_ids, fc_w, fc_b, cls_w, cls_b):
    zsum = jnp.zeros((G2, 128), jnp.float32)
    zcnt = jnp.zeros((G2, CL), jnp.float32)
    ones = jnp.ones((SB, CL), jnp.float32)
    ids = _build_ids(graph_ids)

    parts = []
    for p in range(NPH):
        h4 = _retile(h, p)
        parts.append(_seg_sum_sc(ids[p], h4, zsum, zcnt, ones))

    (s0, c0), (s1, c1) = parts
    s0 = s0.reshape(2, GPAD // 2, D)[:, :G, :]
    s1 = s1.reshape(2, GPAD // 2, D)[:, :G, :]
    c0 = c0.reshape(2, GPAD // 2, 2, CL)[:, :G, 0, :]
    c1 = c1.reshape(2, GPAD // 2, 2, CL)[:, :G, 0, :]
    out = pl.pallas_call(
        _mlp_body,
        out_shape=jax.ShapeDtypeStruct((G, 16), jnp.float32),
    )(s0, s1, c0, c1,
      fc_w, fc_b.reshape(1, 512), cls_w, cls_b.reshape(1, 16))
    return out


# speedup vs baseline: 3.0571x; 1.0287x over previous
import functools

import jax
import jax.numpy as jnp
from jax import lax
from jax.experimental import pallas as pl
from jax.experimental.pallas import tpu as pltpu
from jax.experimental.pallas import tpu_sc as plsc

N = 100000
D = 256
G = 1024
NW = 32
NPH = 1
SLAB = 2560
NSLAB = 40
PH_ROWS = SLAB * NSLAB
NPAD = NPH * PH_ROWS
ROWS_PW = PH_ROWS // NW
SB = 128
NSB = ROWS_PW * 2 // SB
SROWS_W = NSB * SB
G2 = 2 * G
GPAD = G2 + 8
CL = 16
ZSTRIPE = G2 // 16


def _retile_body(h_ref, out_ref):
    x = h_ref[...]
    a = x[:, :128].reshape(SLAB // 8, 8, 128)
    b = x[:, 128:].reshape(SLAB // 8, 8, 128)
    out_ref[...] = jnp.concatenate([a, b], axis=1).reshape(2 * SLAB, 128)


def _retile(h, phase):
    return pl.pallas_call(
        _retile_body,
        grid=(NSLAB,),
        in_specs=[pl.BlockSpec((SLAB, D), lambda i: (phase * NSLAB + i, 0))],
        out_specs=pl.BlockSpec((2 * SLAB, 128), lambda i: (i, 0)),
        out_shape=jax.ShapeDtypeStruct((2 * PH_ROWS, 128), jnp.float32),
    )(h)


_SC_MESH = plsc.VectorSubcoreMesh(core_axis_name="c", subcore_axis_name="s")


@functools.partial(
    pl.kernel,
    mesh=_SC_MESH,
    out_type=[
        jax.ShapeDtypeStruct((2 * GPAD, 128), jnp.float32),
        jax.ShapeDtypeStruct((2 * GPAD, CL), jnp.float32),
    ],
    scratch_types=[
        pltpu.VMEM((NSB, SB), jnp.int32),
        pltpu.VMEM((SB, 128), jnp.float32),
        pltpu.VMEM((SB, 128), jnp.float32),
        pltpu.VMEM((SB, 128), jnp.float32),
        pltpu.VMEM((SB, 128), jnp.float32),
        pltpu.VMEM((SB, 128), jnp.float32),
        pltpu.VMEM((SB, CL), jnp.float32),
        pltpu.VMEM_SHARED((GPAD, 128), jnp.float32),
        pltpu.VMEM_SHARED((GPAD, CL), jnp.float32),
        pltpu.SemaphoreType.DMA,
        pltpu.SemaphoreType.DMA,
        pltpu.SemaphoreType.DMA,
        pltpu.SemaphoreType.DMA,
        pltpu.SemaphoreType.DMA,
        pltpu.SemaphoreType.DMA,
        pltpu.SemaphoreType.DMA,
        pltpu.SemaphoreType.DMA,
        pltpu.SemaphoreType.DMA,
        pltpu.SemaphoreType.DMA,
        pltpu.SemaphoreType.DMA,
    ],
    compiler_params=pltpu.CompilerParams(use_tc_tiling_on_sc=False),
)
def _seg_sum_sc(ids_hbm, h4_hbm, zsum_hbm, zcnt_hbm, ones_hbm,
                sums_hbm, cnts_hbm,
                ids_v, buf0, buf1, buf2, buf3, buf4, ones_v, acc_s, cnt_s,
                ld0, ld1, ld2, ld3, ld4, st0, st1, st2, st3, st4, ctsem):
    cid = lax.axis_index("c")
    sid = lax.axis_index("s")
    wid = sid * 2 + cid
    bufs = (buf0, buf1, buf2, buf3, buf4)
    lds = (ld0, ld1, ld2, ld3, ld4)
    sts = (st0, st1, st2, st3, st4)

    pltpu.sync_copy(ids_hbm.at[wid], ids_v)
    pltpu.sync_copy(ones_hbm, ones_v)
    pltpu.sync_copy(zsum_hbm.at[pl.ds(sid * ZSTRIPE, ZSTRIPE)],
                    acc_s.at[pl.ds(sid * ZSTRIPE, ZSTRIPE)])
    pltpu.sync_copy(zcnt_hbm.at[pl.ds(sid * ZSTRIPE, ZSTRIPE)],
                    cnt_s.at[pl.ds(sid * ZSTRIPE, ZSTRIPE)])
    plsc.subcore_barrier()

    base = wid * SROWS_W

    def h_src(b):
        return h4_hbm.at[pl.ds(base + b * SB, SB)]

    def start_scat(b, k):
        pltpu.async_copy(bufs[k], acc_s.at[ids_v.at[b]], sts[k], add=True)

    def wait_scat(b, k):
        pltpu.make_async_copy(bufs[k], acc_s.at[ids_v.at[b]], sts[k]).wait()

    def cnt_fire(b, carry):
        pltpu.async_copy(ones_v, cnt_s.at[ids_v.at[b]], ctsem, add=True)
        return carry

    lax.fori_loop(0, NSB, cnt_fire, 0)

    pltpu.async_copy(h_src(0), bufs[0], lds[0])
    pltpu.async_copy(h_src(1), bufs[1], lds[1])

    def group(g, carry):
        for k in range(5):
            b = g * 5 + k
            kn = (k + 2) % 5
            @pl.when(b >= 3)
            def _():
                wait_scat(b - 3, kn)
            @pl.when(b + 2 < NSB)
            def _():
                pltpu.async_copy(h_src(b + 2), bufs[kn], lds[kn])
            pltpu.make_async_copy(h_src(b), bufs[k], lds[k]).wait()
            start_scat(b, k)
        return carry

    assert NSB % 5 == 0
    lax.fori_loop(0, NSB // 5, group, 0)

    wait_scat(NSB - 3, (NSB - 3) % 5)
    wait_scat(NSB - 2, (NSB - 2) % 5)
    wait_scat(NSB - 1, (NSB - 1) % 5)

    def cnt_drain(b, carry):
        pltpu.make_async_copy(ones_v, cnt_s.at[ids_v.at[b]], ctsem).wait()
        return carry

    lax.fori_loop(0, NSB, cnt_drain, 0)
    plsc.subcore_barrier()

    pltpu.sync_copy(acc_s.at[pl.ds(sid * ZSTRIPE, ZSTRIPE)],
                    sums_hbm.at[pl.ds(cid * GPAD + sid * ZSTRIPE, ZSTRIPE)])
    pltpu.sync_copy(cnt_s.at[pl.ds(sid * ZSTRIPE, ZSTRIPE)],
                    cnts_hbm.at[pl.ds(cid * GPAD + sid * ZSTRIPE, ZSTRIPE)])


def _mlp_body(s0_ref, c0_ref, fcw_ref, fcb_ref,
              clsw_ref, clsb_ref, out_ref):
    sums = s0_ref[0] + s0_ref[1]
    cnt = c0_ref[0] + c0_ref[1]
    cnt0 = jnp.maximum(cnt[:, 0:1], 1.0)
    gf = sums / cnt0
    hidden = jnp.maximum(jnp.dot(gf, fcw_ref[...]) + fcb_ref[...], 0.0)
    out_ref[...] = jnp.dot(hidden, clsw_ref[...]) + clsb_ref[...]


def _build_ids(graph_ids):
    gid = graph_ids.astype(jnp.int32)
    gidp = jnp.pad(gid, (0, NPAD - N), constant_values=G)
    base = (gidp * 2).reshape(NPH, NW, NSB, 8, 1, 8)
    idx = base + jnp.arange(2, dtype=jnp.int32).reshape(1, 1, 1, 1, 2, 1)
    return idx.reshape(NPH, NW, NSB, SB)


def kernel(h, graph_ids, fc_w, fc_b, cls_w, cls_b):
    zsum = jnp.zeros((G2, 128), jnp.float32)
    zcnt = jnp.zeros((G2, CL), jnp.float32)
    ones = jnp.ones((SB, CL), jnp.float32)
    ids = _build_ids(graph_ids)

    parts = []
    for p in range(NPH):
        h4 = _retile(h, p)
        parts.append(_seg_sum_sc(ids[p], h4, zsum, zcnt, ones))

    ((s0, c0),) = parts
    s0 = s0.reshape(2, GPAD // 2, D)[:, :G, :]
    c0 = c0.reshape(2, GPAD // 2, 2, CL)[:, :G, 0, :]
    out = pl.pallas_call(
        _mlp_body,
        out_shape=jax.ShapeDtypeStruct((G, 16), jnp.float32),
    )(s0, c0,
      fc_w, fc_b.reshape(1, 512), cls_w, cls_b.reshape(1, 16))
    return out


# speedup vs baseline: 3.2426x; 1.0607x over previous
import functools

import jax
import jax.numpy as jnp
import numpy as np
from jax import lax
from jax.experimental import pallas as pl
from jax.experimental.pallas import tpu as pltpu
from jax.experimental.pallas import tpu_sc as plsc

N = 100000
D = 256
G = 1024
NW = 32
ROWS_PER_W = N // NW
BP = 64
NBLK = 50
LOAD_PER_W = NBLK * BP
GPAD = G + 8
CL = 16
STRIPE = G // 16


_SC_MESH = plsc.VectorSubcoreMesh(core_axis_name="c", subcore_axis_name="s")


@functools.partial(
    pl.kernel,
    mesh=_SC_MESH,
    out_type=[
        jax.ShapeDtypeStruct((2 * G, D), jnp.float32),
        jax.ShapeDtypeStruct((2 * G, CL), jnp.float32),
    ],
    scratch_types=[
        pltpu.VMEM((NBLK, BP), jnp.int32),
        pltpu.VMEM((BP, D), jnp.float32),
        pltpu.VMEM((BP, D), jnp.float32),
        pltpu.VMEM((BP, D), jnp.float32),
        pltpu.VMEM((BP, D), jnp.float32),
        pltpu.VMEM((BP, D), jnp.float32),
        pltpu.VMEM((BP, CL), jnp.float32),
        pltpu.VMEM_SHARED((GPAD, D), jnp.float32),
        pltpu.VMEM_SHARED((GPAD, CL), jnp.float32),
        pltpu.SemaphoreType.DMA,
        pltpu.SemaphoreType.DMA,
        pltpu.SemaphoreType.DMA,
        pltpu.SemaphoreType.DMA,
        pltpu.SemaphoreType.DMA,
        pltpu.SemaphoreType.DMA,
        pltpu.SemaphoreType.DMA,
        pltpu.SemaphoreType.DMA,
        pltpu.SemaphoreType.DMA,
        pltpu.SemaphoreType.DMA,
        pltpu.SemaphoreType.DMA,
    ],
    compiler_params=pltpu.CompilerParams(use_tc_tiling_on_sc=False),
)
def _seg_sum_sc(ids_hbm, h_hbm, zsum_hbm, zcnt_hbm, ones_hbm,
                sums_hbm, cnts_hbm,
                ids_v, buf0, buf1, buf2, buf3, buf4, ones_v, acc_s, cnt_s,
                ld0, ld1, ld2, ld3, ld4, st0, st1, st2, st3, st4, ctsem):
    cid = lax.axis_index("c")
    sid = lax.axis_index("s")
    wid = sid * 2 + cid
    bufs = (buf0, buf1, buf2, buf3, buf4)
    lds = (ld0, ld1, ld2, ld3, ld4)
    sts = (st0, st1, st2, st3, st4)

    pltpu.sync_copy(ids_hbm.at[wid], ids_v)
    pltpu.sync_copy(ones_hbm, ones_v)
    pltpu.sync_copy(zsum_hbm.at[pl.ds(sid * STRIPE, STRIPE)],
                    acc_s.at[pl.ds(sid * STRIPE, STRIPE)])
    pltpu.sync_copy(zcnt_hbm.at[pl.ds(sid * STRIPE, STRIPE)],
                    cnt_s.at[pl.ds(sid * STRIPE, STRIPE)])
    plsc.subcore_barrier()

    row0 = jnp.minimum(wid * ROWS_PER_W // 8 * 8, N - LOAD_PER_W)

    def h_src(b):
        return h_hbm.at[pl.ds(row0 + b * BP, BP)]

    def start_scat(b, k):
        pltpu.async_copy(bufs[k], acc_s.at[ids_v.at[b]], sts[k], add=True)

    def wait_scat(b, k):
        pltpu.make_async_copy(bufs[k], acc_s.at[ids_v.at[b]], sts[k]).wait()

    def cnt_fire(b, carry):
        pltpu.async_copy(ones_v, cnt_s.at[ids_v.at[b]], ctsem, add=True)
        return carry

    lax.fori_loop(0, NBLK, cnt_fire, 0)

    pltpu.async_copy(h_src(0), bufs[0], lds[0])
    pltpu.async_copy(h_src(1), bufs[1], lds[1])

    def group(g, carry):
        for k in range(5):
            b = g * 5 + k
            kn = (k + 2) % 5
            @pl.when(b >= 3)
            def _():
                wait_scat(b - 3, kn)
            @pl.when(b + 2 < NBLK)
            def _():
                pltpu.async_copy(h_src(b + 2), bufs[kn], lds[kn])
            pltpu.make_async_copy(h_src(b), bufs[k], lds[k]).wait()
            start_scat(b, k)
        return carry

    assert NBLK % 5 == 0
    lax.fori_loop(0, NBLK // 5, group, 0)

    wait_scat(NBLK - 3, (NBLK - 3) % 5)
    wait_scat(NBLK - 2, (NBLK - 2) % 5)
    wait_scat(NBLK - 1, (NBLK - 1) % 5)

    def cnt_drain(b, carry):
        pltpu.make_async_copy(ones_v, cnt_s.at[ids_v.at[b]], ctsem).wait()
        return carry

    lax.fori_loop(0, NBLK, cnt_drain, 0)
    plsc.subcore_barrier()

    pltpu.sync_copy(acc_s.at[pl.ds(sid * STRIPE, STRIPE)],
                    sums_hbm.at[pl.ds(cid * G + sid * STRIPE, STRIPE)])
    pltpu.sync_copy(cnt_s.at[pl.ds(sid * STRIPE, STRIPE)],
                    cnts_hbm.at[pl.ds(cid * G + sid * STRIPE, STRIPE)])


def _mlp_body(sums_ref, cnts_ref, fcw_ref, fcb_ref, clsw_ref, clsb_ref,
              out_ref):
    sums = sums_ref[0] + sums_ref[1]
    cnt = cnts_ref[0] + cnts_ref[1]
    cnt0 = jnp.maximum(cnt[:, 0:1], 1.0)
    gf = sums / cnt0
    hidden = jnp.maximum(jnp.dot(gf, fcw_ref[...]) + fcb_ref[...], 0.0)
    out_ref[...] = jnp.dot(hidden, clsw_ref[...]) + clsb_ref[...]


_STARTS = [min(w * ROWS_PER_W // 8 * 8, N - LOAD_PER_W) for w in range(NW)]
_OWNED = np.stack([
    (np.arange(s, s + LOAD_PER_W) >= w * ROWS_PER_W)
    & (np.arange(s, s + LOAD_PER_W) < (w + 1) * ROWS_PER_W)
    for w, s in enumerate(_STARTS)
])


def _build_ids(graph_ids):
    gid = graph_ids.astype(jnp.int32)
    wins = jnp.stack([lax.slice(gid, (s,), (s + LOAD_PER_W,))
                      for s in _STARTS])
    ids = jnp.where(_OWNED, wins, G)
    return ids.reshape(NW, NBLK, BP)


def kernel(h, graph_ids, fc_w, fc_b, cls_w, cls_b):
    ids = _build_ids(graph_ids)
    zsum = jnp.zeros((G, D), jnp.float32)
    zcnt = jnp.zeros((G, CL), jnp.float32)
    ones = jnp.ones((BP, CL), jnp.float32)

    sums2, cnts2 = _seg_sum_sc(ids, h, zsum, zcnt, ones)

    out = pl.pallas_call(
        _mlp_body,
        out_shape=jax.ShapeDtypeStruct((G, 16), jnp.float32),
    )(sums2.reshape(2, G, D), cnts2.reshape(2, G, CL),
      fc_w, fc_b.reshape(1, 512), cls_w, cls_b.reshape(1, 16))
    return out


# speedup vs baseline: 3.2546x; 1.0037x over previous
import functools

import jax
import jax.numpy as jnp
import numpy as np
from jax import lax
from jax.experimental import pallas as pl
from jax.experimental.pallas import tpu as pltpu
from jax.experimental.pallas import tpu_sc as plsc

N = 100000
D = 256
G = 1024
NW = 32
ROWS_PER_W = N // NW
BP = 64
NBLK = 50
LOAD_PER_W = NBLK * BP
GPAD = G + 8
CL = 16
STRIPE = G // 16


_SC_MESH = plsc.VectorSubcoreMesh(core_axis_name="c", subcore_axis_name="s")


@functools.partial(
    pl.kernel,
    mesh=_SC_MESH,
    out_type=[
        jax.ShapeDtypeStruct((2 * G, D), jnp.float32),
        jax.ShapeDtypeStruct((2 * G, CL), jnp.float32),
    ],
    scratch_types=[
        pltpu.VMEM((NBLK, BP), jnp.int32),
        pltpu.VMEM((BP, D), jnp.float32),
        pltpu.VMEM((BP, D), jnp.float32),
        pltpu.VMEM((BP, D), jnp.float32),
        pltpu.VMEM((BP, D), jnp.float32),
        pltpu.VMEM((BP, D), jnp.float32),
        pltpu.VMEM((BP, CL), jnp.float32),
        pltpu.VMEM_SHARED((GPAD, D), jnp.float32),
        pltpu.VMEM_SHARED((GPAD, CL), jnp.float32),
        pltpu.SemaphoreType.DMA,
        pltpu.SemaphoreType.DMA,
        pltpu.SemaphoreType.DMA,
        pltpu.SemaphoreType.DMA,
        pltpu.SemaphoreType.DMA,
        pltpu.SemaphoreType.DMA,
        pltpu.SemaphoreType.DMA,
        pltpu.SemaphoreType.DMA,
        pltpu.SemaphoreType.DMA,
        pltpu.SemaphoreType.DMA,
        pltpu.SemaphoreType.DMA,
    ],
    compiler_params=pltpu.CompilerParams(use_tc_tiling_on_sc=False),
)
def _seg_sum_sc(ids_hbm, h_hbm, zsum_hbm, zcnt_hbm, ones_hbm,
                sums_hbm, cnts_hbm,
                ids_v, buf0, buf1, buf2, buf3, buf4, ones_v, acc_s, cnt_s,
                ld0, ld1, ld2, ld3, ld4, st0, st1, st2, st3, st4, ctsem):
    cid = lax.axis_index("c")
    sid = lax.axis_index("s")
    wid = sid * 2 + cid
    bufs = (buf0, buf1, buf2, buf3, buf4)
    lds = (ld0, ld1, ld2, ld3, ld4)
    sts = (st0, st1, st2, st3, st4)

    pltpu.sync_copy(ids_hbm.at[wid], ids_v)
    pltpu.sync_copy(ones_hbm, ones_v)
    pltpu.sync_copy(zsum_hbm.at[pl.ds(sid * STRIPE, STRIPE)],
                    acc_s.at[pl.ds(sid * STRIPE, STRIPE)])
    pltpu.sync_copy(zcnt_hbm.at[pl.ds(sid * STRIPE, STRIPE)],
                    cnt_s.at[pl.ds(sid * STRIPE, STRIPE)])
    plsc.subcore_barrier()

    row0 = jnp.minimum(wid * ROWS_PER_W // 8 * 8, N - LOAD_PER_W)

    def h_src(b):
        return h_hbm.at[pl.ds(row0 + b * BP, BP)]

    def start_scat(b, k):
        pltpu.async_copy(bufs[k], acc_s.at[ids_v.at[b]], sts[k], add=True)

    def wait_scat(b, k):
        pltpu.make_async_copy(bufs[k], acc_s.at[ids_v.at[b]], sts[k]).wait()

    def cnt_fire(b, carry):
        pltpu.async_copy(ones_v, cnt_s.at[ids_v.at[b]], ctsem, add=True)
        return carry

    lax.fori_loop(0, NBLK, cnt_fire, 0)

    pltpu.async_copy(h_src(0), bufs[0], lds[0])
    pltpu.async_copy(h_src(1), bufs[1], lds[1])

    def group(g, carry):
        for k in range(5):
            b = g * 5 + k
            kn = (k + 2) % 5
            @pl.when(b >= 3)
            def _():
                wait_scat(b - 3, kn)
            @pl.when(b + 2 < NBLK)
            def _():
                pltpu.async_copy(h_src(b + 2), bufs[kn], lds[kn])
            pltpu.make_async_copy(h_src(b), bufs[k], lds[k]).wait()
            start_scat(b, k)
        return carry

    assert NBLK % 5 == 0
    lax.fori_loop(0, NBLK // 5, group, 0)

    wait_scat(NBLK - 3, (NBLK - 3) % 5)
    wait_scat(NBLK - 2, (NBLK - 2) % 5)
    wait_scat(NBLK - 1, (NBLK - 1) % 5)

    def cnt_drain(b, carry):
        pltpu.make_async_copy(ones_v, cnt_s.at[ids_v.at[b]], ctsem).wait()
        return carry

    lax.fori_loop(0, NBLK, cnt_drain, 0)
    plsc.subcore_barrier()

    pltpu.sync_copy(acc_s.at[pl.ds(sid * STRIPE, STRIPE)],
                    sums_hbm.at[pl.ds(cid * G + sid * STRIPE, STRIPE)])
    pltpu.sync_copy(cnt_s.at[pl.ds(sid * STRIPE, STRIPE)],
                    cnts_hbm.at[pl.ds(cid * G + sid * STRIPE, STRIPE)])


def _mlp_body(sums_ref, cnts_ref, fcw_ref, fcb_ref, clsw_ref, clsb_ref,
              out_ref):
    sums = sums_ref[:G] + sums_ref[G:]
    cnt = cnts_ref[:G] + cnts_ref[G:]
    cnt0 = jnp.maximum(cnt[:, 0:1], 1.0)
    gf = sums / cnt0
    hidden = jnp.maximum(jnp.dot(gf, fcw_ref[...]) + fcb_ref[...], 0.0)
    out_ref[...] = jnp.dot(hidden, clsw_ref[...]) + clsb_ref[...]


_STARTS = [min(w * ROWS_PER_W // 8 * 8, N - LOAD_PER_W) for w in range(NW)]
_OWNED = np.stack([
    (np.arange(s, s + LOAD_PER_W) >= w * ROWS_PER_W)
    & (np.arange(s, s + LOAD_PER_W) < (w + 1) * ROWS_PER_W)
    for w, s in enumerate(_STARTS)
])


def _build_ids(graph_ids):
    gid = graph_ids.astype(jnp.int32)
    wins = jnp.stack([lax.slice(gid, (s,), (s + LOAD_PER_W,))
                      for s in _STARTS])
    ids = jnp.where(_OWNED, wins, G)
    return ids.reshape(NW, NBLK, BP)


def kernel(h, graph_ids, fc_w, fc_b, cls_w, cls_b):
    ids = _build_ids(graph_ids)
    zsum = jnp.zeros((G, D), jnp.float32)
    zcnt = jnp.zeros((G, CL), jnp.float32)
    ones = jnp.ones((BP, CL), jnp.float32)

    sums2, cnts2 = _seg_sum_sc(ids, h, zsum, zcnt, ones)

    out = pl.pallas_call(
        _mlp_body,
        out_shape=jax.ShapeDtypeStruct((G, 16), jnp.float32),
    )(sums2, cnts2,
      fc_w, fc_b.reshape(1, 512), cls_w, cls_b.reshape(1, 16))
    return out
